# baseline (speedup 1.0000x reference)
import functools

import jax
import jax.numpy as jnp
from jax import lax
from jax.experimental import pallas as pl
from jax.experimental.pallas import tpu as pltpu
from jax.experimental.pallas import tpu_sc as plsc

N = 10000
E = 320000
D = 128
NC = 2
NS = 16
EPT = E // (NC * NS)
CHUNK = 80
ITERS = EPT // CHUNK
DCHUNK = 80
DITERS = EPT // DCHUNK
NPAD = 10240
RPT = NPAD // NS
DEGW = 128
_INV_S = 1.0 / (1.0 + 1e-5) ** 0.5

_mesh = plsc.VectorSubcoreMesh(core_axis_name="c", subcore_axis_name="s")


@functools.partial(
    pl.kernel,
    mesh=_mesh,
    out_type=jax.ShapeDtypeStruct((NC, NPAD, D), jnp.float32),
    scratch_types=[
        pltpu.VMEM_SHARED((NPAD, D), jnp.float32),
        pltpu.VMEM((ITERS, CHUNK), jnp.int32),
        pltpu.VMEM((CHUNK, D), jnp.float32),
        pltpu.SemaphoreType.DMA,
        pltpu.SemaphoreType.DMA,
    ],
)
def _deg_kernel(dst_hbm, zeros_hbm, ones_hbm, degp_hbm, zdeg, dst_v, ones_v,
                sem_a, sem_b):
    c = lax.axis_index("c")
    s = lax.axis_index("s")
    row0 = s * RPT
    wid = c * NS + s
    pltpu.async_copy(dst_hbm.at[wid], dst_v, sem_a)
    pltpu.async_copy(zeros_hbm.at[pl.ds(row0, RPT), :],
                     zdeg.at[pl.ds(row0, RPT), :], sem_b)
    pltpu.sync_copy(ones_hbm, ones_v)
    pltpu.make_async_copy(dst_hbm.at[wid], dst_v, sem_a).wait()
    pltpu.make_async_copy(zeros_hbm.at[pl.ds(row0, RPT), :],
                          zdeg.at[pl.ds(row0, RPT), :], sem_b).wait()
    plsc.subcore_barrier()

    pltpu.async_copy(ones_v, zdeg.at[dst_v.at[0]], sem_a, add=True)

    def body(j, carry):
        k = 2 * j
        pltpu.async_copy(ones_v, zdeg.at[dst_v.at[k + 1]], sem_b, add=True)
        pltpu.make_async_copy(ones_v, zdeg.at[dst_v.at[k]], sem_a).wait()

        @pl.when(k + 2 < ITERS)
        def _():
            pltpu.async_copy(ones_v, zdeg.at[dst_v.at[k + 2]], sem_a, add=True)

        pltpu.make_async_copy(ones_v, zdeg.at[dst_v.at[k + 1]], sem_b).wait()
        return carry

    lax.fori_loop(0, (ITERS - 1) // 2, body, 0)
    pltpu.make_async_copy(ones_v, zdeg.at[dst_v.at[ITERS - 1]], sem_a).wait()
    plsc.subcore_barrier()
    pltpu.sync_copy(zdeg.at[pl.ds(row0, RPT), :],
                    degp_hbm.at[c, pl.ds(row0, RPT), :])


@functools.partial(
    pl.kernel,
    mesh=_mesh,
    out_type=jax.ShapeDtypeStruct((NC, NPAD, D), jnp.float32),
    scratch_types=[
        pltpu.VMEM_SHARED((NPAD, D), jnp.float32),
        pltpu.VMEM((ITERS, CHUNK), jnp.int32),
        pltpu.VMEM((CHUNK,), jnp.int32),
        pltpu.VMEM((CHUNK,), jnp.int32),
        pltpu.VMEM((CHUNK, D), jnp.float32),
        pltpu.VMEM((CHUNK, D), jnp.float32),
        pltpu.SemaphoreType.DMA,
        pltpu.SemaphoreType.DMA,
        pltpu.SemaphoreType.DMA,
        pltpu.SemaphoreType.DMA,
        pltpu.SemaphoreType.DMA,
        pltpu.SemaphoreType.DMA,
    ],
)
def _agg_kernel(y_hbm, src_hbm, dst_hbm, zeros_hbm, zp_hbm,
                z, dst_v, sb_a, sb_b, rows_a, rows_b,
                ssem_a, ssem_b, gsem_a, gsem_b, scsem_a, scsem_b):
    c = lax.axis_index("c")
    s = lax.axis_index("s")
    row0 = s * RPT
    wid = c * NS + s
    pltpu.async_copy(dst_hbm.at[wid], dst_v, gsem_a)
    pltpu.async_copy(zeros_hbm.at[pl.ds(row0, RPT), :],
                     z.at[pl.ds(row0, RPT), :], gsem_b)
    pltpu.make_async_copy(dst_hbm.at[wid], dst_v, gsem_a).wait()
    pltpu.make_async_copy(zeros_hbm.at[pl.ds(row0, RPT), :],
                          z.at[pl.ds(row0, RPT), :], gsem_b).wait()
    plsc.subcore_barrier()

    pltpu.sync_copy(src_hbm.at[wid, 0], sb_a)
    pltpu.async_copy(y_hbm.at[sb_a], rows_a, gsem_a)
    pltpu.async_copy(src_hbm.at[wid, 1], sb_b, ssem_b)

    def phase(k, sb_p, ssem_p, rows_p, gsem_p, scsem_p,
              sb_q, ssem_q, rows_q, gsem_q, scsem_q):
        pltpu.make_async_copy(src_hbm.at[wid, k + 1], sb_q, ssem_q).wait()

        @pl.when(k >= 1)
        def _():
            pltpu.make_async_copy(rows_q, z.at[dst_v.at[k]], scsem_q).wait()

        pltpu.async_copy(y_hbm.at[sb_q], rows_q, gsem_q)
        pltpu.make_async_copy(y_hbm.at[sb_p], rows_p, gsem_p).wait()
        pltpu.async_copy(rows_p, z.at[dst_v.at[k]], scsem_p, add=True)

        @pl.when(k + 2 < ITERS)
        def _():
            pltpu.async_copy(src_hbm.at[wid, k + 2], sb_p, ssem_p)

    def body(j, carry):
        i = 2 * j
        phase(i, sb_a, ssem_a, rows_a, gsem_a, scsem_a,
              sb_b, ssem_b, rows_b, gsem_b, scsem_b)
        phase(i + 1, sb_b, ssem_b, rows_b, gsem_b, scsem_b,
              sb_a, ssem_a, rows_a, gsem_a, scsem_a)
        return carry

    lax.fori_loop(0, (ITERS - 1) // 2, body, 0)
    pltpu.make_async_copy(y_hbm.at[sb_a], rows_a, gsem_a).wait()
    pltpu.sync_copy(rows_a, z.at[dst_v.at[ITERS - 1]], add=True)
    pltpu.make_async_copy(rows_b, z.at[dst_v.at[ITERS - 2]], scsem_b).wait()

    plsc.subcore_barrier()
    pltpu.sync_copy(z.at[pl.ds(row0, RPT), :],
                    zp_hbm.at[c, pl.ds(row0, RPT), :])


BLK = 1000
GRID = N // BLK

def _tc_first_body(degp_ref, x_ref, w_ref, dinv_ref, y_ref):
    dp = degp_ref[...]
    deg = dp[0] [:, 0:1] + dp[1][:, 0:1] + 1.0
    dinv = lax.rsqrt(deg)
    dinv_ref[...] = dinv
    y_ref[...] = dinv * jnp.dot(x_ref[...], w_ref[...],
                                preferred_element_type=jnp.float32)


_tc_first = pl.pallas_call(
    _tc_first_body,
    grid=(GRID,),
    in_specs=[
        pl.BlockSpec((NC, BLK, D), lambda i: (0, i, 0)),
        pl.BlockSpec((BLK, D), lambda i: (i, 0)),
        pl.BlockSpec((D, D), lambda i: (0, 0)),
    ],
    out_specs=(pl.BlockSpec((BLK, 1), lambda i: (i, 0)),
               pl.BlockSpec((BLK, D), lambda i: (i, 0))),
    out_shape=(jax.ShapeDtypeStruct((N, 1), jnp.float32),
               jax.ShapeDtypeStruct((N, D), jnp.float32)),
)


def _tc_mid_body(zp_ref, y_ref, dinv_ref, b_ref, g_ref, be_ref, w_ref, yn_ref):
    zp = zp_ref[...]
    dinv = dinv_ref[...]
    z = zp[0] + zp[1] + y_ref[...]
    t = dinv * z + b_ref[...]
    t = g_ref[...] * (t * _INV_S) + be_ref[...]
    t = jnp.maximum(t, 0.0)
    yn_ref[...] = dinv * jnp.dot(t, w_ref[...],
                                 preferred_element_type=jnp.float32)


_tc_mid = pl.pallas_call(
    _tc_mid_body,
    grid=(GRID,),
    in_specs=[
        pl.BlockSpec((NC, BLK, D), lambda i: (0, i, 0)),
        pl.BlockSpec((BLK, D), lambda i: (i, 0)),
        pl.BlockSpec((BLK, 1), lambda i: (i, 0)),
        pl.BlockSpec((1, D), lambda i: (0, 0)),
        pl.BlockSpec((1, D), lambda i: (0, 0)),
        pl.BlockSpec((1, D), lambda i: (0, 0)),
        pl.BlockSpec((D, D), lambda i: (0, 0)),
    ],
    out_specs=pl.BlockSpec((BLK, D), lambda i: (i, 0)),
    out_shape=jax.ShapeDtypeStruct((N, D), jnp.float32),
)


def _tc_last_body(zp_ref, y_ref, dinv_ref, b_ref, out_ref):
    zp = zp_ref[...]
    z = zp[0] + zp[1] + y_ref[...]
    out_ref[...] = dinv_ref[...] * z + b_ref[...]


_tc_last = pl.pallas_call(
    _tc_last_body,
    grid=(GRID,),
    in_specs=[
        pl.BlockSpec((NC, BLK, D), lambda i: (0, i, 0)),
        pl.BlockSpec((BLK, D), lambda i: (i, 0)),
        pl.BlockSpec((BLK, 1), lambda i: (i, 0)),
        pl.BlockSpec((1, D), lambda i: (0, 0)),
    ],
    out_specs=pl.BlockSpec((BLK, D), lambda i: (i, 0)),
    out_shape=jax.ShapeDtypeStruct((N, D), jnp.float32),
)


def kernel(x, edge_index, W1, b1, g1, be1, W2, b2, g2, be2, W3, b3):
    src = edge_index[0]
    dst = edge_index[1]
    src3 = src.reshape(NC * NS, ITERS, CHUNK)
    dst3 = dst.reshape(NC * NS, ITERS, CHUNK)
    del src, dst
    zeros_nd = jnp.zeros((NPAD, D), jnp.float32)
    ones_cd = jnp.ones((CHUNK, D), jnp.float32)

    degp = _deg_kernel(dst3, zeros_nd, ones_cd)
    dinv, y1 = _tc_first(degp, x, W1)

    zp1 = _agg_kernel(y1, src3, dst3, zeros_nd)
    y2 = _tc_mid(zp1, y1, dinv, b1.reshape(1, D), g1.reshape(1, D),
                 be1.reshape(1, D), W2)

    zp2 = _agg_kernel(y2, src3, dst3, zeros_nd)
    y3 = _tc_mid(zp2, y2, dinv, b2.reshape(1, D), g2.reshape(1, D),
                 be2.reshape(1, D), W3)

    zp3 = _agg_kernel(y3, src3, dst3, zeros_nd)
    return _tc_last(zp3, y3, dinv, b3.reshape(1, D))


# speedup vs baseline: 1.0490x; 1.0490x over previous
import functools

import jax
import jax.numpy as jnp
from jax import lax
from jax.experimental import pallas as pl
from jax.experimental.pallas import tpu as pltpu
from jax.experimental.pallas import tpu_sc as plsc

N = 10000
E = 320000
D = 128
NC = 2
NS = 16
EPT = E // (NC * NS)
CHUNK = 80
ITERS = EPT // CHUNK
DCHUNK = 80
DITERS = EPT // DCHUNK
NPAD = 10240
RPT = NPAD // NS
DEGW = 128
_INV_S = 1.0 / (1.0 + 1e-5) ** 0.5

_mesh = plsc.VectorSubcoreMesh(core_axis_name="c", subcore_axis_name="s")


@functools.partial(
    pl.kernel,
    mesh=_mesh,
    out_type=jax.ShapeDtypeStruct((NC, NPAD, D), jnp.float32),
    scratch_types=[
        pltpu.VMEM_SHARED((NPAD, D), jnp.float32),
        pltpu.VMEM((ITERS, CHUNK), jnp.int32),
        pltpu.VMEM((CHUNK, D), jnp.float32),
        pltpu.SemaphoreType.DMA,
        pltpu.SemaphoreType.DMA,
    ],
)
def _deg_kernel(dst_hbm, zeros_hbm, ones_hbm, degp_hbm, zdeg, dst_v, ones_v,
                sem_a, sem_b):
    c = lax.axis_index("c")
    s = lax.axis_index("s")
    row0 = s * RPT
    wid = c * NS + s
    pltpu.async_copy(dst_hbm.at[wid], dst_v, sem_a)
    pltpu.async_copy(zeros_hbm.at[pl.ds(row0, RPT), :],
                     zdeg.at[pl.ds(row0, RPT), :], sem_b)
    pltpu.sync_copy(ones_hbm, ones_v)
    pltpu.make_async_copy(dst_hbm.at[wid], dst_v, sem_a).wait()
    pltpu.make_async_copy(zeros_hbm.at[pl.ds(row0, RPT), :],
                          zdeg.at[pl.ds(row0, RPT), :], sem_b).wait()
    plsc.subcore_barrier()

    pltpu.async_copy(ones_v, zdeg.at[dst_v.at[0]], sem_a, add=True)

    def body(j, carry):
        k = 2 * j
        pltpu.async_copy(ones_v, zdeg.at[dst_v.at[k + 1]], sem_b, add=True)
        pltpu.make_async_copy(ones_v, zdeg.at[dst_v.at[k]], sem_a).wait()

        @pl.when(k + 2 < ITERS)
        def _():
            pltpu.async_copy(ones_v, zdeg.at[dst_v.at[k + 2]], sem_a, add=True)

        pltpu.make_async_copy(ones_v, zdeg.at[dst_v.at[k + 1]], sem_b).wait()
        return carry

    lax.fori_loop(0, (ITERS - 1) // 2, body, 0)
    pltpu.make_async_copy(ones_v, zdeg.at[dst_v.at[ITERS - 1]], sem_a).wait()
    plsc.subcore_barrier()
    pltpu.sync_copy(zdeg.at[pl.ds(row0, RPT), :],
                    degp_hbm.at[c, pl.ds(row0, RPT), :])


@functools.partial(
    pl.kernel,
    mesh=_mesh,
    out_type=jax.ShapeDtypeStruct((NC, NPAD, D), jnp.float32),
    scratch_types=[
        pltpu.VMEM_SHARED((NPAD, D), jnp.float32),
        pltpu.VMEM((ITERS, CHUNK), jnp.int32),
        pltpu.VMEM((CHUNK,), jnp.int32),
        pltpu.VMEM((CHUNK,), jnp.int32),
        pltpu.VMEM((CHUNK, D), jnp.float32),
        pltpu.VMEM((CHUNK, D), jnp.float32),
        pltpu.SemaphoreType.DMA,
        pltpu.SemaphoreType.DMA,
        pltpu.SemaphoreType.DMA,
        pltpu.SemaphoreType.DMA,
        pltpu.SemaphoreType.DMA,
        pltpu.SemaphoreType.DMA,
    ],
)
def _agg_kernel(y_hbm, src_hbm, dst_hbm, zeros_hbm, zp_hbm,
                z, dst_v, sb_a, sb_b, rows_a, rows_b,
                ssem_a, ssem_b, gsem_a, gsem_b, scsem_a, scsem_b):
    c = lax.axis_index("c")
    s = lax.axis_index("s")
    row0 = s * RPT
    wid = c * NS + s
    pltpu.async_copy(dst_hbm.at[wid], dst_v, gsem_a)
    pltpu.async_copy(zeros_hbm.at[pl.ds(row0, RPT), :],
                     z.at[pl.ds(row0, RPT), :], gsem_b)
    pltpu.make_async_copy(dst_hbm.at[wid], dst_v, gsem_a).wait()
    pltpu.make_async_copy(zeros_hbm.at[pl.ds(row0, RPT), :],
                          z.at[pl.ds(row0, RPT), :], gsem_b).wait()
    plsc.subcore_barrier()

    pltpu.sync_copy(src_hbm.at[wid, 0], sb_a)
    pltpu.async_copy(y_hbm.at[sb_a], rows_a, gsem_a)
    pltpu.async_copy(src_hbm.at[wid, 1], sb_b, ssem_b)

    def phase(k, sb_p, ssem_p, rows_p, gsem_p, scsem_p,
              sb_q, ssem_q, rows_q, gsem_q, scsem_q):
        pltpu.make_async_copy(src_hbm.at[wid, k + 1], sb_q, ssem_q).wait()

        @pl.when(k >= 1)
        def _():
            pltpu.make_async_copy(rows_q, z.at[dst_v.at[k]], scsem_q).wait()

        pltpu.async_copy(y_hbm.at[sb_q], rows_q, gsem_q)
        pltpu.make_async_copy(y_hbm.at[sb_p], rows_p, gsem_p).wait()
        pltpu.async_copy(rows_p, z.at[dst_v.at[k]], scsem_p, add=True)

        @pl.when(k + 2 < ITERS)
        def _():
            pltpu.async_copy(src_hbm.at[wid, k + 2], sb_p, ssem_p)

    def body(j, carry):
        i = 2 * j
        phase(i, sb_a, ssem_a, rows_a, gsem_a, scsem_a,
              sb_b, ssem_b, rows_b, gsem_b, scsem_b)
        phase(i + 1, sb_b, ssem_b, rows_b, gsem_b, scsem_b,
              sb_a, ssem_a, rows_a, gsem_a, scsem_a)
        return carry

    lax.fori_loop(0, (ITERS - 1) // 2, body, 0)
    pltpu.make_async_copy(y_hbm.at[sb_a], rows_a, gsem_a).wait()
    pltpu.sync_copy(rows_a, z.at[dst_v.at[ITERS - 1]], add=True)
    pltpu.make_async_copy(rows_b, z.at[dst_v.at[ITERS - 2]], scsem_b).wait()

    plsc.subcore_barrier()
    pltpu.sync_copy(z.at[pl.ds(row0, RPT), :],
                    zp_hbm.at[c, pl.ds(row0, RPT), :])


def _tc_first_body(degp_ref, x_ref, w_ref, dinv_ref, y_ref):
    dp = degp_ref[...]
    deg = dp[0][0:N, 0:1] + dp[1][0:N, 0:1] + 1.0
    dinv = lax.rsqrt(deg)
    dinv_ref[...] = dinv
    y_ref[...] = dinv * jnp.dot(x_ref[...], w_ref[...],
                                preferred_element_type=jnp.float32)


_tc_first = pl.pallas_call(
    _tc_first_body,
    out_shape=(jax.ShapeDtypeStruct((N, 1), jnp.float32),
               jax.ShapeDtypeStruct((N, D), jnp.float32)),
)


def _tc_mid_body(zp_ref, y_ref, dinv_ref, b_ref, g_ref, be_ref, w_ref, yn_ref):
    zp = zp_ref[...]
    dinv = dinv_ref[...]
    z = zp[0][0:N, :] + zp[1][0:N, :] + y_ref[...]
    t = dinv * z + b_ref[...]
    t = g_ref[...] * (t * _INV_S) + be_ref[...]
    t = jnp.maximum(t, 0.0)
    yn_ref[...] = dinv * jnp.dot(t, w_ref[...],
                                 preferred_element_type=jnp.float32)


_tc_mid = pl.pallas_call(
    _tc_mid_body,
    out_shape=jax.ShapeDtypeStruct((N, D), jnp.float32),
)


def _tc_last_body(zp_ref, y_ref, dinv_ref, b_ref, out_ref):
    zp = zp_ref[...]
    z = zp[0][0:N, :] + zp[1][0:N, :] + y_ref[...]
    out_ref[...] = dinv_ref[...] * z + b_ref[...]


_tc_last = pl.pallas_call(
    _tc_last_body,
    out_shape=jax.ShapeDtypeStruct((N, D), jnp.float32),
)


def kernel(x, edge_index, W1, b1, g1, be1, W2, b2, g2, be2, W3, b3):
    src = edge_index[0]
    dst = edge_index[1]
    src3 = src.reshape(NC * NS, ITERS, CHUNK)
    dst3 = dst.reshape(NC * NS, ITERS, CHUNK)
    del src, dst
    zeros_nd = jnp.zeros((NPAD, D), jnp.float32)
    ones_cd = jnp.ones((CHUNK, D), jnp.float32)

    degp = _deg_kernel(dst3, zeros_nd, ones_cd)
    dinv, y1 = _tc_first(degp, x, W1)

    zp1 = _agg_kernel(y1, src3, dst3, zeros_nd)
    y2 = _tc_mid(zp1, y1, dinv, b1.reshape(1, D), g1.reshape(1, D),
                 be1.reshape(1, D), W2)

    zp2 = _agg_kernel(y2, src3, dst3, zeros_nd)
    y3 = _tc_mid(zp2, y2, dinv, b2.reshape(1, D), g2.reshape(1, D),
                 be2.reshape(1, D), W3)

    zp3 = _agg_kernel(y3, src3, dst3, zeros_nd)
    return _tc_last(zp3, y3, dinv, b3.reshape(1, D))
